# baseline (speedup 1.0000x reference)
import jax
import jax.numpy as jnp
from jax import lax
from jax.experimental import pallas as pl

_NODE_CONT = {0: [0, 1, 4, 6, 7, 8, 9, 10, 11, 12, 13],
              1: [0, 1, 4, 5, 6, 7, 8, 9, 10, 11, 12, 13],
              2: [1, 2, 4, 5, 6, 7, 8, 9, 10, 11, 12, 13],
              3: [2, 3, 4, 5, 6, 7, 8, 9, 10, 11, 12, 13]}
_NODE_DISC_DIMS = {0: {2: 96, 3: 8, 5: 2}, 1: {2: 4, 3: 22}, 2: {0: 6}, 3: {0: 15, 1: 96}}
_NODE_DISC = {0: [2, 3, 5], 1: [2, 3], 2: [0], 3: [0, 1]}
_EMB_DIM = 256
_NUM_T = 4
_N = 100000
_NF = 14
_R = 1000


def _split_dims(t):
    feats = _NODE_DISC[t]
    n = len(feats)
    per = _EMB_DIM // n
    rem = _EMB_DIM % n
    return [per + (1 if i < rem else 0) for i in range(n)]


def _layout():
    entries = []
    voff = 0
    for t in range(_NUM_T):
        dims = _split_dims(t)
        coff = 0
        for i, f in enumerate(_NODE_DISC[t]):
            vocab = _NODE_DISC_DIMS[t][f]
            entries.append((t, f, voff, vocab, coff, dims[i]))
            voff += vocab
            coff += dims[i]
    return entries, voff


_ENTRIES, _VTOT = _layout()
_DUMMY = 255

_SLOTS = []
for _k in range(max(len(_NODE_DISC[t]) for t in range(_NUM_T)) + 1):
    slot = {}
    for _t in range(_NUM_T):
        ent = [e for e in _ENTRIES if e[0] == _t]
        if _k < len(ent):
            slot[_t] = (ent[_k][1], ent[_k][2])
        elif _k == len(ent):
            slot[_t] = (None, _VTOT + _t)
        else:
            slot[_t] = (None, _DUMMY)
    _SLOTS.append(slot)


def _body(x_ref, nt_ref, e_ref, w_ref, o_ref):
    xb = x_ref[...]
    tt = nt_ref[...]
    xi = xb.astype(jnp.int32)
    iota = lax.broadcasted_iota(jnp.int32, (_R, 256), 1)
    sel = None
    for slot in _SLOTS:
        tgt = None
        for t in range(_NUM_T - 1, -1, -1):
            f, off = slot[t]
            v = (xi[:, f:f + 1] + off) if f is not None else jnp.full((_R, 1), off, jnp.int32)
            tgt = v if tgt is None else jnp.where(tt == t, v, tgt)
        c = iota == tgt
        sel = c if sel is None else sel | c
    onehot = jnp.where(sel, 1.0, 0.0).astype(jnp.bfloat16)
    acc = jnp.dot(onehot, e_ref[...], preferred_element_type=jnp.float32)
    for t in range(_NUM_T):
        xt = jnp.where(tt == t, xb, 0.0).astype(jnp.bfloat16)
        acc = acc + jnp.dot(xt, w_ref[t * _NF:(t + 1) * _NF, :],
                            preferred_element_type=jnp.float32)
    o_ref[...] = acc


def _pack_weights(params):
    rows = []
    for (t, f, voff, vocab, coff, dim) in _ENTRIES:
        tbl = params["emb"][str(t)][str(f)]
        rows.append(jnp.pad(tbl, ((0, 0), (coff, _EMB_DIM - coff - dim))))
    for t in range(_NUM_T):
        rows.append(params["b"][str(t)][None, :])
    rows.append(jnp.zeros((256 - _VTOT - _NUM_T, _EMB_DIM), jnp.float32))
    e = jnp.concatenate(rows, axis=0)

    wrows = []
    for t in range(_NUM_T):
        wt_t = params["W"][str(t)].T
        wt_pad = jnp.concatenate([wt_t, jnp.zeros((1, _EMB_DIM), jnp.float32)], axis=0)
        gather = [len(_NODE_CONT[t])] * _NF
        for p_i, f in enumerate(_NODE_CONT[t]):
            gather[f] = p_i
        wrows.append(jnp.take(wt_pad, jnp.array(gather), axis=0))
    wf = jnp.concatenate(wrows, axis=0)
    return e.astype(jnp.bfloat16), wf.astype(jnp.bfloat16)


def kernel(x, node_types, params):
    e, wf = _pack_weights(params)
    nt = node_types.astype(jnp.int32).reshape(_N, 1)
    grid = _N // _R
    out = pl.pallas_call(
        _body,
        grid=(grid,),
        in_specs=[
            pl.BlockSpec((_R, _NF), lambda i: (i, 0)),
            pl.BlockSpec((_R, 1), lambda i: (i, 0)),
            pl.BlockSpec((256, _EMB_DIM), lambda i: (0, 0)),
            pl.BlockSpec((_NUM_T * _NF, _EMB_DIM), lambda i: (0, 0)),
        ],
        out_specs=pl.BlockSpec((_R, _EMB_DIM), lambda i: (i, 0)),
        out_shape=jax.ShapeDtypeStruct((_N, _EMB_DIM), jnp.float32),
    )(x, nt, e, wf)
    return out


# speedup vs baseline: 2.7543x; 2.7543x over previous
import jax
import jax.numpy as jnp
from jax import lax
from jax.experimental import pallas as pl

_NODE_CONT = {0: [0, 1, 4, 6, 7, 8, 9, 10, 11, 12, 13],
              1: [0, 1, 4, 5, 6, 7, 8, 9, 10, 11, 12, 13],
              2: [1, 2, 4, 5, 6, 7, 8, 9, 10, 11, 12, 13],
              3: [2, 3, 4, 5, 6, 7, 8, 9, 10, 11, 12, 13]}
_NODE_DISC_DIMS = {0: {2: 96, 3: 8, 5: 2}, 1: {2: 4, 3: 22}, 2: {0: 6}, 3: {0: 15, 1: 96}}
_NODE_DISC = {0: [2, 3, 5], 1: [2, 3], 2: [0], 3: [0, 1]}
_EMB_DIM = 256
_NUM_T = 4
_N = 100000
_NF = 14
_R = 1024
_NBLK = -(-_N // _R)
_NPAD = _NBLK * _R


def _split_dims(t):
    feats = _NODE_DISC[t]
    n = len(feats)
    per = _EMB_DIM // n
    rem = _EMB_DIM % n
    return [per + (1 if i < rem else 0) for i in range(n)]


def _layout():
    entries = []
    voff = 0
    for t in range(_NUM_T):
        dims = _split_dims(t)
        coff = 0
        for i, f in enumerate(_NODE_DISC[t]):
            vocab = _NODE_DISC_DIMS[t][f]
            entries.append((t, f, voff, vocab, coff, dims[i]))
            voff += vocab
            coff += dims[i]
    return entries, voff


_ENTRIES, _VTOT = _layout()
_DUMMY = 255

_SLOTS = []
for _k in range(max(len(_NODE_DISC[t]) for t in range(_NUM_T)) + 1):
    _slot = {}
    for _t in range(_NUM_T):
        _ent = [e for e in _ENTRIES if e[0] == _t]
        if _k < len(_ent):
            _slot[_t] = (_ent[_k][1], _ent[_k][2])
        elif _k == len(_ent):
            _slot[_t] = (None, _VTOT + _t)
        else:
            _slot[_t] = (None, _DUMMY)
    _SLOTS.append(_slot)

_CONTRACT0 = (((0,), (0,)), ((), ()))


def _body(xt_ref, nt_ref, e_ref, w_ref, o_ref):
    xtb = xt_ref[...]
    ttt = nt_ref[...]
    xit = xtb.astype(jnp.int32)
    iota = lax.broadcasted_iota(jnp.int32, (256, _R), 0)
    sel = None
    for slot in _SLOTS:
        tgt = None
        for t in range(_NUM_T - 1, -1, -1):
            f, off = slot[t]
            v = (xit[f:f + 1, :] + off) if f is not None else jnp.full((1, _R), off, jnp.int32)
            tgt = v if tgt is None else jnp.where(ttt == t, v, tgt)
        c = iota == tgt
        sel = c if sel is None else sel | c
    onehot_t = jnp.where(sel, 1.0, 0.0).astype(jnp.bfloat16)
    acc = lax.dot_general(onehot_t, e_ref[...], _CONTRACT0,
                          preferred_element_type=jnp.float32)
    p_t = jnp.concatenate(
        [jnp.where(ttt == t, xtb, 0.0) for t in range(_NUM_T)], axis=0)
    acc = acc + lax.dot_general(p_t.astype(jnp.bfloat16), w_ref[...], _CONTRACT0,
                                preferred_element_type=jnp.float32)
    o_ref[...] = acc


def _pack_weights(params):
    rows = []
    for (t, f, voff, vocab, coff, dim) in _ENTRIES:
        tbl = params["emb"][str(t)][str(f)]
        rows.append(jnp.pad(tbl, ((0, 0), (coff, _EMB_DIM - coff - dim))))
    for t in range(_NUM_T):
        rows.append(params["b"][str(t)][None, :])
    rows.append(jnp.zeros((256 - _VTOT - _NUM_T, _EMB_DIM), jnp.float32))
    e = jnp.concatenate(rows, axis=0)

    wrows = []
    for t in range(_NUM_T):
        wt_t = params["W"][str(t)].T
        wt_pad = jnp.concatenate([wt_t, jnp.zeros((1, _EMB_DIM), jnp.float32)], axis=0)
        gather = [len(_NODE_CONT[t])] * 16
        for p_i, f in enumerate(_NODE_CONT[t]):
            gather[f] = p_i
        wrows.append(jnp.take(wt_pad, jnp.array(gather), axis=0))
    wf = jnp.concatenate(wrows, axis=0)
    return e.astype(jnp.bfloat16), wf.astype(jnp.bfloat16)


def kernel(x, node_types, params):
    e, wf = _pack_weights(params)
    xt = jnp.pad(x, ((0, _NPAD - _N), (0, 16 - _NF))).T
    nt = jnp.pad(node_types.astype(jnp.int32), (0, _NPAD - _N)).reshape(1, _NPAD)
    grid = _NBLK
    out = pl.pallas_call(
        _body,
        grid=(grid,),
        in_specs=[
            pl.BlockSpec((16, _R), lambda i: (0, i)),
            pl.BlockSpec((1, _R), lambda i: (0, i)),
            pl.BlockSpec((256, _EMB_DIM), lambda i: (0, 0)),
            pl.BlockSpec((64, _EMB_DIM), lambda i: (0, 0)),
        ],
        out_specs=pl.BlockSpec((_R, _EMB_DIM), lambda i: (i, 0)),
        out_shape=jax.ShapeDtypeStruct((_N, _EMB_DIM), jnp.float32),
    )(xt, nt, e, wf)
    return out


# speedup vs baseline: 3.7634x; 1.3664x over previous
import jax
import jax.numpy as jnp
from jax import lax
from jax.experimental import pallas as pl

_NODE_CONT = {0: [0, 1, 4, 6, 7, 8, 9, 10, 11, 12, 13],
              1: [0, 1, 4, 5, 6, 7, 8, 9, 10, 11, 12, 13],
              2: [1, 2, 4, 5, 6, 7, 8, 9, 10, 11, 12, 13],
              3: [2, 3, 4, 5, 6, 7, 8, 9, 10, 11, 12, 13]}
_NODE_DISC_DIMS = {0: {2: 96, 3: 8, 5: 2}, 1: {2: 4, 3: 22}, 2: {0: 6}, 3: {0: 15, 1: 96}}
_NODE_DISC = {0: [2, 3, 5], 1: [2, 3], 2: [0], 3: [0, 1]}
_EMB_DIM = 256
_NUM_T = 4
_N = 100000
_NF = 14
_R = 2048
_NBLK = -(-_N // _R)
_NPAD = _NBLK * _R


def _split_dims(t):
    feats = _NODE_DISC[t]
    n = len(feats)
    per = _EMB_DIM // n
    rem = _EMB_DIM % n
    return [per + (1 if i < rem else 0) for i in range(n)]


def _layout():
    entries = []
    voff = 0
    for t in range(_NUM_T):
        dims = _split_dims(t)
        coff = 0
        for i, f in enumerate(_NODE_DISC[t]):
            vocab = _NODE_DISC_DIMS[t][f]
            entries.append((t, f, voff, vocab, coff, dims[i]))
            voff += vocab
            coff += dims[i]
    return entries, voff


_ENTRIES, _VTOT = _layout()
_DUMMY = 255

_SLOTS = []
for _k in range(max(len(_NODE_DISC[t]) for t in range(_NUM_T))):
    _slot = {}
    for _t in range(_NUM_T):
        _ent = [e for e in _ENTRIES if e[0] == _t]
        if _k < len(_ent):
            _slot[_t] = (_ent[_k][1], _ent[_k][2])
        else:
            _slot[_t] = (None, _DUMMY)
    _SLOTS.append(_slot)

_CONTRACT0 = (((0,), (0,)), ((), ()))


def _body(xt_ref, nt_ref, e_ref, w_ref, o_ref):
    xtb = xt_ref[...]
    ttt = nt_ref[...]
    xit = xtb.astype(jnp.int32)
    iota = lax.broadcasted_iota(jnp.int32, (256, _R), 0)
    sel = None
    for slot in _SLOTS:
        tgt = None
        for t in range(_NUM_T - 1, -1, -1):
            f, off = slot[t]
            v = (xit[f:f + 1, :] + off) if f is not None else jnp.full((1, _R), off, jnp.int32)
            tgt = v if tgt is None else jnp.where(ttt == t, v, tgt)
        c = iota == tgt
        sel = c if sel is None else sel | c
    onehot_t = jnp.where(sel, 1.0, 0.0).astype(jnp.bfloat16)
    acc = lax.dot_general(onehot_t, e_ref[...], _CONTRACT0,
                          preferred_element_type=jnp.float32)
    p_t = jnp.concatenate(
        [jnp.where(ttt == t, xtb, 0.0) for t in range(_NUM_T)], axis=0)
    acc = acc + lax.dot_general(p_t.astype(jnp.bfloat16), w_ref[...], _CONTRACT0,
                                preferred_element_type=jnp.float32)
    o_ref[...] = acc


def _pack_weights(params):
    rows = []
    for (t, f, voff, vocab, coff, dim) in _ENTRIES:
        tbl = params["emb"][str(t)][str(f)] + params["b"][str(t)][coff:coff + dim]
        rows.append(jnp.pad(tbl, ((0, 0), (coff, _EMB_DIM - coff - dim))))
    rows.append(jnp.zeros((256 - _VTOT, _EMB_DIM), jnp.float32))
    e = jnp.concatenate(rows, axis=0)

    wrows = []
    for t in range(_NUM_T):
        wt_t = params["W"][str(t)].T
        wt_pad = jnp.concatenate([wt_t, jnp.zeros((1, _EMB_DIM), jnp.float32)], axis=0)
        gather = [len(_NODE_CONT[t])] * 16
        for p_i, f in enumerate(_NODE_CONT[t]):
            gather[f] = p_i
        wrows.append(jnp.take(wt_pad, jnp.array(gather), axis=0))
    wf = jnp.concatenate(wrows, axis=0)
    return e.astype(jnp.bfloat16), wf.astype(jnp.bfloat16)


def kernel(x, node_types, params):
    e, wf = _pack_weights(params)
    xt = jnp.pad(x, ((0, _NPAD - _N), (0, 16 - _NF))).T
    nt = jnp.pad(node_types.astype(jnp.int32), (0, _NPAD - _N)).reshape(1, _NPAD)
    grid = _NBLK
    out = pl.pallas_call(
        _body,
        grid=(grid,),
        in_specs=[
            pl.BlockSpec((16, _R), lambda i: (0, i)),
            pl.BlockSpec((1, _R), lambda i: (0, i)),
            pl.BlockSpec((256, _EMB_DIM), lambda i: (0, 0)),
            pl.BlockSpec((64, _EMB_DIM), lambda i: (0, 0)),
        ],
        out_specs=pl.BlockSpec((_R, _EMB_DIM), lambda i: (i, 0)),
        out_shape=jax.ShapeDtypeStruct((_N, _EMB_DIM), jnp.float32),
    )(xt, nt, e, wf)
    return out
